# initial kernel scaffold (unmeasured)
import jax
import jax.numpy as jnp
from jax import lax
from jax.experimental import pallas as pl
from jax.experimental.pallas import tpu as pltpu

N_DEV = 4


def kernel(ids, E):
    V_shard, D = E.shape
    T = ids.shape[0]
    my_pos = lax.axis_index("i")

    local = ids - my_pos * V_shard
    valid = (local >= 0) & (local < V_shard)
    safe = jnp.where(valid, local, 0)
    partial = jnp.take(E, safe, axis=0) * valid[:, None].astype(E.dtype)

    chunk = T // N_DEV
    n_hops = 2 * (N_DEV - 1)

    def body(x_ref, out_ref, comm_ref, send_sems, recv_sems, credit_sem):
        my = lax.axis_index("i")
        left = (my - 1) % N_DEV
        right = (my + 1) % N_DEV

        barrier_sem = pltpu.get_barrier_semaphore()
        for nbr in [left, right]:
            pl.semaphore_signal(
                barrier_sem, inc=1,
                device_id=(nbr,), device_id_type=pl.DeviceIdType.MESH,
            )
        pl.semaphore_wait(barrier_sem, 2)

        out_ref[...] = x_ref[...]

        for h in range(n_hops):
            slot = h % 2
            if h < N_DEV - 1:
                send_c = (my - h) % N_DEV
                recv_c = (my - h - 1) % N_DEV
            else:
                s = h - (N_DEV - 1)
                send_c = (my + 1 - s) % N_DEV
                recv_c = (my - s) % N_DEV

            if h >= 2:
                pl.semaphore_wait(credit_sem, 1)

            rdma = pltpu.make_async_remote_copy(
                src_ref=out_ref.at[pl.ds(send_c * chunk, chunk), :],
                dst_ref=comm_ref.at[slot],
                send_sem=send_sems.at[h],
                recv_sem=recv_sems.at[h],
                device_id=(right,),
                device_id_type=pl.DeviceIdType.MESH,
            )
            rdma.start()
            rdma.wait()

            if h < N_DEV - 1:
                out_ref[pl.ds(recv_c * chunk, chunk), :] = (
                    out_ref[pl.ds(recv_c * chunk, chunk), :] + comm_ref[slot]
                )
            else:
                out_ref[pl.ds(recv_c * chunk, chunk), :] = comm_ref[slot]

            pl.semaphore_signal(
                credit_sem, inc=1,
                device_id=(left,), device_id_type=pl.DeviceIdType.MESH,
            )

        pl.semaphore_wait(credit_sem, 2)

    return pl.pallas_call(
        body,
        out_shape=jax.ShapeDtypeStruct((T, D), jnp.float32),
        in_specs=[pl.BlockSpec(memory_space=pltpu.VMEM)],
        out_specs=pl.BlockSpec(memory_space=pltpu.VMEM),
        scratch_shapes=[
            pltpu.VMEM((2, chunk, D), jnp.float32),
            pltpu.SemaphoreType.DMA((n_hops,)),
            pltpu.SemaphoreType.DMA((n_hops,)),
            pltpu.SemaphoreType.REGULAR,
        ],
        input_output_aliases={0: 0},
        compiler_params=pltpu.CompilerParams(collective_id=0),
    )(partial)


# baseline (device time: 3360216 ns/iter reference)
import jax
import jax.numpy as jnp
from jax import lax
from jax.experimental import pallas as pl
from jax.experimental.pallas import tpu as pltpu

N_DEV = 4


def kernel(ids, E):
    V_shard, D = E.shape
    T = ids.shape[0]
    my_pos = lax.axis_index("i")

    local = ids - my_pos * V_shard
    valid = (local >= 0) & (local < V_shard)
    safe = jnp.where(valid, local, 0)
    partial = jnp.take(E, safe, axis=0) * valid[:, None].astype(E.dtype)

    chunk = T // N_DEV
    n_hops = 2 * (N_DEV - 1)

    def body(x_ref, out_ref, comm_ref, work_ref,
             send_sems, recv_sems, local_sem, credit_sem):
        my = lax.axis_index("i")
        left = (my - 1) % N_DEV
        right = (my + 1) % N_DEV

        barrier_sem = pltpu.get_barrier_semaphore()
        for nbr in [left, right]:
            pl.semaphore_signal(
                barrier_sem, inc=1,
                device_id=(nbr,), device_id_type=pl.DeviceIdType.MESH,
            )
        pl.semaphore_wait(barrier_sem, 2)

        def rows(c):
            return pl.ds(c * chunk, chunk)

        for h in range(n_hops):
            slot = h % 2
            if h < N_DEV - 1:
                send_c = (my - h) % N_DEV
                recv_c = (my - h - 1) % N_DEV
            else:
                s = h - (N_DEV - 1)
                send_c = (my + 1 - s) % N_DEV
                recv_c = (my - s) % N_DEV

            if h >= 2:
                pl.semaphore_wait(credit_sem, 1)

            rdma = pltpu.make_async_remote_copy(
                src_ref=out_ref.at[rows(send_c), :],
                dst_ref=comm_ref.at[slot],
                send_sem=send_sems.at[h],
                recv_sem=recv_sems.at[h],
                device_id=(right,),
                device_id_type=pl.DeviceIdType.MESH,
            )
            rdma.start()
            rdma.wait()

            if h < N_DEV - 1:
                cp_in = pltpu.make_async_copy(
                    out_ref.at[rows(recv_c), :], work_ref, local_sem)
                cp_in.start()
                cp_in.wait()
                work_ref[...] = work_ref[...] + comm_ref[slot]
                cp_out = pltpu.make_async_copy(
                    work_ref, out_ref.at[rows(recv_c), :], local_sem)
            else:
                cp_out = pltpu.make_async_copy(
                    comm_ref.at[slot], out_ref.at[rows(recv_c), :], local_sem)
            cp_out.start()
            cp_out.wait()

            pl.semaphore_signal(
                credit_sem, inc=1,
                device_id=(left,), device_id_type=pl.DeviceIdType.MESH,
            )

        pl.semaphore_wait(credit_sem, 2)

    return pl.pallas_call(
        body,
        out_shape=jax.ShapeDtypeStruct((T, D), jnp.float32),
        in_specs=[pl.BlockSpec(memory_space=pl.ANY)],
        out_specs=pl.BlockSpec(memory_space=pl.ANY),
        scratch_shapes=[
            pltpu.VMEM((2, chunk, D), jnp.float32),
            pltpu.VMEM((chunk, D), jnp.float32),
            pltpu.SemaphoreType.DMA((n_hops,)),
            pltpu.SemaphoreType.DMA((n_hops,)),
            pltpu.SemaphoreType.DMA,
            pltpu.SemaphoreType.REGULAR,
        ],
        input_output_aliases={0: 0},
        compiler_params=pltpu.CompilerParams(collective_id=0),
    )(partial)


# device time: 752382 ns/iter; 4.4661x vs baseline; 4.4661x over previous
import jax
import jax.numpy as jnp
from jax import lax
from jax.experimental import pallas as pl
from jax.experimental.pallas import tpu as pltpu

N_DEV = 4


def kernel(ids, E):
    V_shard, D = E.shape
    T = ids.shape[0]
    my_pos = lax.axis_index("i")

    local = ids - my_pos * V_shard
    valid = (local >= 0) & (local < V_shard)
    safe = jnp.where(valid, local, 0).astype(jnp.int32)
    maskf = valid.astype(jnp.float32)[:, None]

    chunk = T // N_DEV
    n_hops = 2 * (N_DEV - 1)

    def body(safe_ref, mask_ref, e_ref, out_ref, comm_ref, work_ref,
             send_sems, recv_sems, gather_sem, local_sem, credit_sem):
        my = lax.axis_index("i")
        left = (my - 1) % N_DEV
        right = (my + 1) % N_DEV

        barrier_sem = pltpu.get_barrier_semaphore()
        for nbr in [left, right]:
            pl.semaphore_signal(
                barrier_sem, inc=1,
                device_id=(nbr,), device_id_type=pl.DeviceIdType.MESH,
            )
        pl.semaphore_wait(barrier_sem, 2)

        def rows(c):
            return pl.ds(c * chunk, chunk)

        def gather_chunk(c):
            base = c * chunk

            def issue(t, _):
                idx = safe_ref[base + t]
                pltpu.make_async_copy(
                    e_ref.at[pl.ds(idx, 1), :],
                    work_ref.at[pl.ds(t, 1), :],
                    gather_sem,
                ).start()
                return 0

            lax.fori_loop(0, chunk, issue, 0)

            def drain(t, _):
                pltpu.make_async_copy(
                    e_ref.at[pl.ds(0, 1), :],
                    work_ref.at[pl.ds(0, 1), :],
                    gather_sem,
                ).wait()
                return 0

            lax.fori_loop(0, chunk, drain, 0)
            work_ref[...] = work_ref[...] * mask_ref[rows(c), :]
            cp = pltpu.make_async_copy(
                work_ref, out_ref.at[rows(c), :], local_sem)
            cp.start()
            cp.wait()

        for c in range(N_DEV):
            gather_chunk(c)

        for h in range(n_hops):
            slot = h % 2
            if h < N_DEV - 1:
                send_c = (my - h) % N_DEV
                recv_c = (my - h - 1) % N_DEV
            else:
                s = h - (N_DEV - 1)
                send_c = (my + 1 - s) % N_DEV
                recv_c = (my - s) % N_DEV

            if h >= 2:
                pl.semaphore_wait(credit_sem, 1)

            rdma = pltpu.make_async_remote_copy(
                src_ref=out_ref.at[rows(send_c), :],
                dst_ref=comm_ref.at[slot],
                send_sem=send_sems.at[h],
                recv_sem=recv_sems.at[h],
                device_id=(right,),
                device_id_type=pl.DeviceIdType.MESH,
            )
            rdma.start()
            rdma.wait()

            if h < N_DEV - 1:
                cp_in = pltpu.make_async_copy(
                    out_ref.at[rows(recv_c), :], work_ref, local_sem)
                cp_in.start()
                cp_in.wait()
                work_ref[...] = work_ref[...] + comm_ref[slot]
                cp_out = pltpu.make_async_copy(
                    work_ref, out_ref.at[rows(recv_c), :], local_sem)
            else:
                cp_out = pltpu.make_async_copy(
                    comm_ref.at[slot], out_ref.at[rows(recv_c), :], local_sem)
            cp_out.start()
            cp_out.wait()

            pl.semaphore_signal(
                credit_sem, inc=1,
                device_id=(left,), device_id_type=pl.DeviceIdType.MESH,
            )

        pl.semaphore_wait(credit_sem, 2)

    return pl.pallas_call(
        body,
        out_shape=jax.ShapeDtypeStruct((T, D), jnp.float32),
        in_specs=[
            pl.BlockSpec(memory_space=pltpu.SMEM),
            pl.BlockSpec(memory_space=pltpu.VMEM),
            pl.BlockSpec(memory_space=pl.ANY),
        ],
        out_specs=pl.BlockSpec(memory_space=pl.ANY),
        scratch_shapes=[
            pltpu.VMEM((2, chunk, D), jnp.float32),
            pltpu.VMEM((chunk, D), jnp.float32),
            pltpu.SemaphoreType.DMA((n_hops,)),
            pltpu.SemaphoreType.DMA((n_hops,)),
            pltpu.SemaphoreType.DMA,
            pltpu.SemaphoreType.DMA,
            pltpu.SemaphoreType.REGULAR,
        ],
        compiler_params=pltpu.CompilerParams(collective_id=0),
    )(safe, maskf, E)


# device time: 444999 ns/iter; 7.5511x vs baseline; 1.6907x over previous
import jax
import jax.numpy as jnp
from jax import lax
from jax.experimental import pallas as pl
from jax.experimental.pallas import tpu as pltpu

N_DEV = 4
UNROLL = 8


def kernel(ids, E):
    V_shard, D = E.shape
    T = ids.shape[0]
    my_pos = lax.axis_index("i")

    local = ids - my_pos * V_shard
    valid = (local >= 0) & (local < V_shard)
    safe = jnp.where(valid, local, 0).astype(jnp.int32)
    maskf = valid.astype(jnp.float32)[:, None]

    chunk = T // N_DEV
    hw = D // 2
    n_hops = 2 * (N_DEV - 1)

    def body(safe_ref, mask_ref, e_ref, out_ref, part_ref,
             comm_cw, comm_ccw,
             cw_send_sems, cw_recv_sems, ccw_send_sems, ccw_recv_sems,
             gather_sem, out_sem, credit_cw, credit_ccw):
        my = lax.axis_index("i")
        left = (my - 1) % N_DEV
        right = (my + 1) % N_DEV

        barrier_sem = pltpu.get_barrier_semaphore()
        for nbr in [left, right]:
            pl.semaphore_signal(
                barrier_sem, inc=1,
                device_id=(nbr,), device_id_type=pl.DeviceIdType.MESH,
            )
        pl.semaphore_wait(barrier_sem, 2)

        def rows(c):
            return pl.ds(c * chunk, chunk)

        def issue(i, _):
            t = i * UNROLL
            for u in range(UNROLL):
                idx = safe_ref[t + u]
                pltpu.make_async_copy(
                    e_ref.at[pl.ds(idx, 1), :],
                    part_ref.at[pl.ds(t + u, 1), :],
                    gather_sem,
                ).start()
            return 0

        lax.fori_loop(0, T // UNROLL, issue, 0)

        def drain(i, _):
            for _u in range(UNROLL):
                pltpu.make_async_copy(
                    e_ref.at[pl.ds(0, 1), :],
                    part_ref.at[pl.ds(0, 1), :],
                    gather_sem,
                ).wait()
            return 0

        lax.fori_loop(0, T // UNROLL, drain, 0)
        part_ref[...] = part_ref[...] * mask_ref[...]

        n_out = 0
        for h in range(n_hops):
            slot = h % 2
            if h < N_DEV - 1:
                cw_send = (my - h) % N_DEV
                cw_recv = (my - h - 1) % N_DEV
                ccw_send = (my + h) % N_DEV
                ccw_recv = (my + h + 1) % N_DEV
            else:
                s = h - (N_DEV - 1)
                cw_send = (my + 1 - s) % N_DEV
                cw_recv = (my - s) % N_DEV
                ccw_send = (my - 1 + s) % N_DEV
                ccw_recv = (my + s) % N_DEV

            if h >= 2:
                pl.semaphore_wait(credit_cw, 1)
                pl.semaphore_wait(credit_ccw, 1)

            rdma_cw = pltpu.make_async_remote_copy(
                src_ref=part_ref.at[rows(cw_send), pl.ds(0, hw)],
                dst_ref=comm_cw.at[slot],
                send_sem=cw_send_sems.at[h],
                recv_sem=cw_recv_sems.at[h],
                device_id=(right,),
                device_id_type=pl.DeviceIdType.MESH,
            )
            rdma_ccw = pltpu.make_async_remote_copy(
                src_ref=part_ref.at[rows(ccw_send), pl.ds(hw, hw)],
                dst_ref=comm_ccw.at[slot],
                send_sem=ccw_send_sems.at[h],
                recv_sem=ccw_recv_sems.at[h],
                device_id=(left,),
                device_id_type=pl.DeviceIdType.MESH,
            )
            rdma_cw.start()
            rdma_ccw.start()
            rdma_cw.wait()
            rdma_ccw.wait()

            if h < N_DEV - 1:
                part_ref[rows(cw_recv), pl.ds(0, hw)] = (
                    part_ref[rows(cw_recv), pl.ds(0, hw)] + comm_cw[slot]
                )
                part_ref[rows(ccw_recv), pl.ds(hw, hw)] = (
                    part_ref[rows(ccw_recv), pl.ds(hw, hw)] + comm_ccw[slot]
                )
            else:
                part_ref[rows(cw_recv), pl.ds(0, hw)] = comm_cw[slot]
                part_ref[rows(ccw_recv), pl.ds(hw, hw)] = comm_ccw[slot]

            pl.semaphore_signal(
                credit_cw, inc=1,
                device_id=(left,), device_id_type=pl.DeviceIdType.MESH,
            )
            pl.semaphore_signal(
                credit_ccw, inc=1,
                device_id=(right,), device_id_type=pl.DeviceIdType.MESH,
            )

            if h >= N_DEV - 2:
                if h == N_DEV - 2:
                    done = [(cw_recv, 0), (ccw_recv, hw)]
                else:
                    done = [(cw_recv, 0), (ccw_recv, hw)]
                for c, col in done:
                    pltpu.make_async_copy(
                        part_ref.at[rows(c), pl.ds(col, hw)],
                        out_ref.at[rows(c), pl.ds(col, hw)],
                        out_sem,
                    ).start()
                    n_out += 1

        pl.semaphore_wait(credit_cw, 2)
        pl.semaphore_wait(credit_ccw, 2)

        for _ in range(n_out):
            pltpu.make_async_copy(
                part_ref.at[rows(0), pl.ds(0, hw)],
                out_ref.at[rows(0), pl.ds(0, hw)],
                out_sem,
            ).wait()

    return pl.pallas_call(
        body,
        out_shape=jax.ShapeDtypeStruct((T, D), jnp.float32),
        in_specs=[
            pl.BlockSpec(memory_space=pltpu.SMEM),
            pl.BlockSpec(memory_space=pltpu.VMEM),
            pl.BlockSpec(memory_space=pl.ANY),
        ],
        out_specs=pl.BlockSpec(memory_space=pl.ANY),
        scratch_shapes=[
            pltpu.VMEM((T, D), jnp.float32),
            pltpu.VMEM((2, chunk, hw), jnp.float32),
            pltpu.VMEM((2, chunk, hw), jnp.float32),
            pltpu.SemaphoreType.DMA((n_hops,)),
            pltpu.SemaphoreType.DMA((n_hops,)),
            pltpu.SemaphoreType.DMA((n_hops,)),
            pltpu.SemaphoreType.DMA((n_hops,)),
            pltpu.SemaphoreType.DMA,
            pltpu.SemaphoreType.DMA,
            pltpu.SemaphoreType.REGULAR,
            pltpu.SemaphoreType.REGULAR,
        ],
        compiler_params=pltpu.CompilerParams(
            collective_id=0,
            vmem_limit_bytes=60 * 1024 * 1024,
        ),
    )(safe, maskf, E)


# device time: 345276 ns/iter; 9.7320x vs baseline; 1.2888x over previous
import jax
import jax.numpy as jnp
from jax import lax
from jax.experimental import pallas as pl
from jax.experimental.pallas import tpu as pltpu

N_DEV = 4
UNROLL = 4


def kernel(ids, E):
    V_shard, D = E.shape
    T = ids.shape[0]
    my_pos = lax.axis_index("i")

    local = (ids - my_pos * V_shard).astype(jnp.int32)
    valid = (local >= 0) & (local < V_shard)
    counts = jnp.sum(
        valid.reshape(N_DEV, T // N_DEV).astype(jnp.int32), axis=1)

    chunk = T // N_DEV
    hw = D // 2
    n_hops = 2 * (N_DEV - 1)

    def body(local_ref, counts_ref, e_ref, out_ref, part_ref,
             comm_cw, comm_ccw,
             cw_send_sems, cw_recv_sems, ccw_send_sems, ccw_recv_sems,
             gather_sem, gather_sem2, out_sem, credit_cw, credit_ccw):
        my = lax.axis_index("i")
        left = (my - 1) % N_DEV
        right = (my + 1) % N_DEV
        diag = (my + 2) % N_DEV

        def rows(c):
            return pl.ds(c * chunk, chunk)

        part_ref[...] = jnp.zeros((T, D), jnp.float32)

        def issue_chunk(c, sem):
            base = c * chunk

            def b(i, _):
                t = base + i * UNROLL
                for u in range(UNROLL):
                    idx = local_ref[t + u]

                    @pl.when((idx >= 0) & (idx < V_shard))
                    def _():
                        pltpu.make_async_copy(
                            e_ref.at[pl.ds(idx, 1), :],
                            part_ref.at[pl.ds(t + u, 1), :],
                            sem,
                        ).start()
                return 0

            lax.fori_loop(0, chunk // UNROLL, b, 0)

        def drain(n, sem):
            def b(i, _):
                pltpu.make_async_copy(
                    e_ref.at[pl.ds(0, 1), :],
                    part_ref.at[pl.ds(0, 1), :],
                    sem,
                ).wait()
                return 0

            lax.fori_loop(0, n, b, 0)

        issue_chunk(my, gather_sem)

        barrier_sem = pltpu.get_barrier_semaphore()
        for nbr in [left, right]:
            pl.semaphore_signal(
                barrier_sem, inc=1,
                device_id=(nbr,), device_id_type=pl.DeviceIdType.MESH,
            )
        pl.semaphore_wait(barrier_sem, 2)
        drain(counts_ref[my], gather_sem)

        n_out = 0
        for h in range(n_hops):
            slot = h % 2
            if h < N_DEV - 1:
                cw_send = (my - h) % N_DEV
                cw_recv = (my - h - 1) % N_DEV
                ccw_send = (my + h) % N_DEV
                ccw_recv = (my + h + 1) % N_DEV
            else:
                s = h - (N_DEV - 1)
                cw_send = (my + 1 - s) % N_DEV
                cw_recv = (my - s) % N_DEV
                ccw_send = (my - 1 + s) % N_DEV
                ccw_recv = (my + s) % N_DEV

            if h >= 2:
                pl.semaphore_wait(credit_cw, 1)
                pl.semaphore_wait(credit_ccw, 1)

            rdma_cw = pltpu.make_async_remote_copy(
                src_ref=part_ref.at[rows(cw_send), pl.ds(0, hw)],
                dst_ref=comm_cw.at[slot],
                send_sem=cw_send_sems.at[h],
                recv_sem=cw_recv_sems.at[h],
                device_id=(right,),
                device_id_type=pl.DeviceIdType.MESH,
            )
            rdma_ccw = pltpu.make_async_remote_copy(
                src_ref=part_ref.at[rows(ccw_send), pl.ds(hw, hw)],
                dst_ref=comm_ccw.at[slot],
                send_sem=ccw_send_sems.at[h],
                recv_sem=ccw_recv_sems.at[h],
                device_id=(left,),
                device_id_type=pl.DeviceIdType.MESH,
            )
            rdma_cw.start()
            rdma_ccw.start()

            if h == 0:
                issue_chunk(left, gather_sem)
                issue_chunk(right, gather_sem)
                issue_chunk(diag, gather_sem2)
                drain(counts_ref[left] + counts_ref[right], gather_sem)
            elif h == 1:
                drain(counts_ref[diag], gather_sem2)

            rdma_cw.wait()
            rdma_ccw.wait()

            if h < N_DEV - 1:
                part_ref[rows(cw_recv), pl.ds(0, hw)] = (
                    part_ref[rows(cw_recv), pl.ds(0, hw)] + comm_cw[slot]
                )
                part_ref[rows(ccw_recv), pl.ds(hw, hw)] = (
                    part_ref[rows(ccw_recv), pl.ds(hw, hw)] + comm_ccw[slot]
                )
            else:
                part_ref[rows(cw_recv), pl.ds(0, hw)] = comm_cw[slot]
                part_ref[rows(ccw_recv), pl.ds(hw, hw)] = comm_ccw[slot]

            pl.semaphore_signal(
                credit_cw, inc=1,
                device_id=(left,), device_id_type=pl.DeviceIdType.MESH,
            )
            pl.semaphore_signal(
                credit_ccw, inc=1,
                device_id=(right,), device_id_type=pl.DeviceIdType.MESH,
            )

            if h >= N_DEV - 2:
                for c, col in [(cw_recv, 0), (ccw_recv, hw)]:
                    pltpu.make_async_copy(
                        part_ref.at[rows(c), pl.ds(col, hw)],
                        out_ref.at[rows(c), pl.ds(col, hw)],
                        out_sem,
                    ).start()
                    n_out += 1

        pl.semaphore_wait(credit_cw, 2)
        pl.semaphore_wait(credit_ccw, 2)

        for _ in range(n_out):
            pltpu.make_async_copy(
                part_ref.at[rows(0), pl.ds(0, hw)],
                out_ref.at[rows(0), pl.ds(0, hw)],
                out_sem,
            ).wait()

    return pl.pallas_call(
        body,
        out_shape=jax.ShapeDtypeStruct((T, D), jnp.float32),
        in_specs=[
            pl.BlockSpec(memory_space=pltpu.SMEM),
            pl.BlockSpec(memory_space=pltpu.SMEM),
            pl.BlockSpec(memory_space=pl.ANY),
        ],
        out_specs=pl.BlockSpec(memory_space=pl.ANY),
        scratch_shapes=[
            pltpu.VMEM((T, D), jnp.float32),
            pltpu.VMEM((2, chunk, hw), jnp.float32),
            pltpu.VMEM((2, chunk, hw), jnp.float32),
            pltpu.SemaphoreType.DMA((n_hops,)),
            pltpu.SemaphoreType.DMA((n_hops,)),
            pltpu.SemaphoreType.DMA((n_hops,)),
            pltpu.SemaphoreType.DMA((n_hops,)),
            pltpu.SemaphoreType.DMA,
            pltpu.SemaphoreType.DMA,
            pltpu.SemaphoreType.DMA,
            pltpu.SemaphoreType.REGULAR,
            pltpu.SemaphoreType.REGULAR,
        ],
        compiler_params=pltpu.CompilerParams(
            collective_id=0,
            vmem_limit_bytes=60 * 1024 * 1024,
        ),
    )(local, counts, E)
